# baseline (speedup 1.0000x reference)
import functools

import jax
import jax.numpy as jnp
from jax import lax
from jax.experimental import pallas as pl
from jax.experimental.pallas import tpu as pltpu
from jax.experimental.pallas import tpu_sc as plsc

_LANES = 16
_EMB_DIM = 64
_BLK = 128


def _dot_body(uid_hbm, iid_hbm, user_t, item_t, out_hbm,
              idx_v, ublk_v, iblk_v, out_v, sem_u, sem_i):
    cid = lax.axis_index("c")
    sid = lax.axis_index("s")

    @pl.when(jnp.logical_and(cid == 0, sid == 0))
    def _():
        cp_uid = pltpu.async_copy(uid_hbm, idx_v.at[pl.ds(0, 1)], sem_u)
        cp_iid = pltpu.async_copy(iid_hbm, idx_v.at[pl.ds(8, 1)], sem_i)
        cp_uid.wait()
        cp_iid.wait()
        iv = idx_v[...]
        u = iv[0]
        it = iv[8]
        n_user = user_t.shape[1]
        n_item = item_t.shape[1]
        cu = pl.multiple_of(jnp.minimum((u // _BLK) * _BLK, n_user - _BLK),
                            _BLK)
        ci = pl.multiple_of(jnp.minimum((it // _BLK) * _BLK, n_item - _BLK),
                            _BLK)
        cp_u = pltpu.async_copy(user_t.at[:, pl.ds(cu, _BLK)], ublk_v, sem_u)
        cp_i = pltpu.async_copy(item_t.at[:, pl.ds(ci, _BLK)], iblk_v, sem_i)
        lane_u = u - cu
        lane_i = it - ci
        base_u = pl.multiple_of((lane_u // _LANES) * _LANES, _LANES)
        base_i = pl.multiple_of((lane_i // _LANES) * _LANES, _LANES)
        sub_u = jnp.full((_LANES,), lane_u % _LANES, dtype=jnp.int32)
        sub_i = jnp.full((_LANES,), lane_i % _LANES, dtype=jnp.int32)
        cp_u.wait()
        cp_i.wait()
        def step(d, acc):
            bu = ublk_v[d, pl.ds(base_u, _LANES)].at[sub_u].get(
                mode="promise_in_bounds")
            bi = iblk_v[d, pl.ds(base_i, _LANES)].at[sub_i].get(
                mode="promise_in_bounds")
            return acc + bu * bi

        acc = lax.fori_loop(0, _EMB_DIM, step,
                            jnp.zeros((_LANES,), jnp.float32))
        out_v[...] = acc
        pltpu.sync_copy(out_v, out_hbm)


@jax.jit
def _mf_dot(uid, iid, user_t, item_t):
    call = pl.kernel(
        _dot_body,
        out_type=jax.ShapeDtypeStruct((_LANES,), jnp.float32),
        mesh=plsc.VectorSubcoreMesh(core_axis_name="c", subcore_axis_name="s",
                                    num_cores=1),
        scratch_types=[
            pltpu.VMEM((_LANES,), jnp.int32),
            pltpu.VMEM((_EMB_DIM, _BLK), jnp.float32),
            pltpu.VMEM((_EMB_DIM, _BLK), jnp.float32),
            pltpu.VMEM((_LANES,), jnp.float32),
            pltpu.SemaphoreType.DMA,
            pltpu.SemaphoreType.DMA,
        ],
    )
    return call(uid, iid, user_t, item_t)


def kernel(user_id, item_id, user_table, item_table):
    uid = jnp.reshape(user_id.astype(jnp.int32), (1,))
    iid = jnp.reshape(item_id.astype(jnp.int32), (1,))
    out = _mf_dot(uid, iid, user_table.T, item_table.T)
    return out[0]


# speedup vs baseline: 1.0152x; 1.0152x over previous
import functools

import jax
import jax.numpy as jnp
from jax import lax
from jax.experimental import pallas as pl
from jax.experimental.pallas import tpu as pltpu
from jax.experimental.pallas import tpu_sc as plsc

_LANES = 16
_EMB_DIM = 64
_BLK = 128


def _dot_body(uid_hbm, iid_hbm, user_t, item_t, out_hbm,
              idx_v, ublk_v, iblk_v, out_v, sem_u, sem_i):
    cid = lax.axis_index("c")
    sid = lax.axis_index("s")

    @pl.when(jnp.logical_and(cid == 0, sid == 0))
    def _():
        cp_uid = pltpu.async_copy(uid_hbm, idx_v.at[pl.ds(0, 1)], sem_u)
        cp_iid = pltpu.async_copy(iid_hbm, idx_v.at[pl.ds(8, 1)], sem_i)
        cp_uid.wait()
        cp_iid.wait()
        iv = idx_v[...]
        u = iv[0]
        it = iv[8]
        n_user = user_t.shape[1]
        n_item = item_t.shape[1]
        cu = pl.multiple_of(jnp.minimum((u // _BLK) * _BLK, n_user - _BLK),
                            _BLK)
        ci = pl.multiple_of(jnp.minimum((it // _BLK) * _BLK, n_item - _BLK),
                            _BLK)
        cp_u = pltpu.async_copy(user_t.at[:, pl.ds(cu, _BLK)], ublk_v, sem_u)
        cp_i = pltpu.async_copy(item_t.at[:, pl.ds(ci, _BLK)], iblk_v, sem_i)
        lane_u = u - cu
        lane_i = it - ci
        base_u = pl.multiple_of((lane_u // _LANES) * _LANES, _LANES)
        base_i = pl.multiple_of((lane_i // _LANES) * _LANES, _LANES)
        sub_u = jnp.full((_LANES,), lane_u % _LANES, dtype=jnp.int32)
        sub_i = jnp.full((_LANES,), lane_i % _LANES, dtype=jnp.int32)
        cp_u.wait()
        cp_i.wait()
        def step(d, acc):
            bu = ublk_v[d, pl.ds(base_u, _LANES)].at[sub_u].get(
                mode="promise_in_bounds")
            bi = iblk_v[d, pl.ds(base_i, _LANES)].at[sub_i].get(
                mode="promise_in_bounds")
            return acc + bu * bi

        acc = lax.fori_loop(0, _EMB_DIM, step,
                            jnp.zeros((_LANES,), jnp.float32))
        out_v[...] = acc
        pltpu.sync_copy(out_v, out_hbm)


@jax.jit
def _mf_dot(uid, iid, user_t, item_t):
    call = pl.kernel(
        _dot_body,
        out_type=jax.ShapeDtypeStruct((_LANES,), jnp.float32),
        mesh=plsc.VectorSubcoreMesh(core_axis_name="c", subcore_axis_name="s",
                                    num_cores=1, num_subcores=1),
        scratch_types=[
            pltpu.VMEM((_LANES,), jnp.int32),
            pltpu.VMEM((_EMB_DIM, _BLK), jnp.float32),
            pltpu.VMEM((_EMB_DIM, _BLK), jnp.float32),
            pltpu.VMEM((_LANES,), jnp.float32),
            pltpu.SemaphoreType.DMA,
            pltpu.SemaphoreType.DMA,
        ],
    )
    return call(uid, iid, user_t, item_t)


def kernel(user_id, item_id, user_table, item_table):
    uid = jnp.reshape(user_id.astype(jnp.int32), (1,))
    iid = jnp.reshape(item_id.astype(jnp.int32), (1,))
    out = _mf_dot(uid, iid, user_table.T, item_table.T)
    return out[0]


# speedup vs baseline: 1.0163x; 1.0011x over previous
import jax
import jax.numpy as jnp
from jax import lax
from jax.experimental import pallas as pl
from jax.experimental.pallas import tpu as pltpu
from jax.experimental.pallas import tpu_sc as plsc

_LANES = 16
_EMB_DIM = 64
_BLK = 128


def _dot_body(uid_hbm, iid_hbm, user_t, item_t, out_hbm,
              idx_v, ublk_v, iblk_v, out_v, sem_u, sem_i):
    cid = lax.axis_index("c")
    sid = lax.axis_index("s")

    @pl.when(jnp.logical_and(cid == 0, sid == 0))
    def _():
        cp_uid = pltpu.async_copy(uid_hbm, idx_v.at[pl.ds(0, 1)], sem_u)
        cp_iid = pltpu.async_copy(iid_hbm, idx_v.at[pl.ds(8, 1)], sem_i)
        cp_uid.wait()
        cp_iid.wait()
        iv = idx_v[...]
        u = iv[0]
        it = iv[8]
        n_user = user_t.shape[1]
        n_item = item_t.shape[1]
        cu = pl.multiple_of(jnp.minimum((u // _BLK) * _BLK, n_user - _BLK),
                            _BLK)
        ci = pl.multiple_of(jnp.minimum((it // _BLK) * _BLK, n_item - _BLK),
                            _BLK)
        cp_u = pltpu.async_copy(user_t.at[:, pl.ds(cu, _BLK)], ublk_v, sem_u)
        cp_i = pltpu.async_copy(item_t.at[:, pl.ds(ci, _BLK)], iblk_v, sem_i)
        lane_u = u - cu
        lane_i = it - ci
        base_u = pl.multiple_of((lane_u // _LANES) * _LANES, _LANES)
        base_i = pl.multiple_of((lane_i // _LANES) * _LANES, _LANES)
        sub_u = jnp.full((_LANES,), lane_u % _LANES, dtype=jnp.int32)
        sub_i = jnp.full((_LANES,), lane_i % _LANES, dtype=jnp.int32)
        cp_u.wait()
        cp_i.wait()
        def step(d, acc):
            bu = ublk_v[d, pl.ds(base_u, _LANES)].at[sub_u].get(
                mode="promise_in_bounds")
            bi = iblk_v[d, pl.ds(base_i, _LANES)].at[sub_i].get(
                mode="promise_in_bounds")
            return acc + bu * bi

        acc = lax.fori_loop(0, _EMB_DIM, step,
                            jnp.zeros((_LANES,), jnp.float32))
        out_v[...] = acc
        pltpu.sync_copy(out_v, out_hbm)


@jax.jit
def _mf_dot(uid, iid, user_t, item_t):
    call = pl.kernel(
        _dot_body,
        out_type=jax.ShapeDtypeStruct((_LANES,), jnp.float32),
        mesh=plsc.VectorSubcoreMesh(core_axis_name="c", subcore_axis_name="s",
                                    num_cores=1, num_subcores=1),
        scratch_types=[
            pltpu.VMEM((_LANES,), jnp.int32),
            pltpu.VMEM((_EMB_DIM, _BLK), jnp.float32),
            pltpu.VMEM((_EMB_DIM, _BLK), jnp.float32),
            pltpu.VMEM((_LANES,), jnp.float32),
            pltpu.SemaphoreType.DMA,
            pltpu.SemaphoreType.DMA,
        ],
    )
    return call(uid, iid, user_t, item_t)


def kernel(user_id, item_id, user_table, item_table):
    uid = jnp.reshape(user_id.astype(jnp.int32), (1,))
    iid = jnp.reshape(item_id.astype(jnp.int32), (1,))
    out = _mf_dot(uid, iid, user_table.T, item_table.T)
    return out[0]
